# baseline (speedup 1.0000x reference)
import functools

import jax
import jax.numpy as jnp
import numpy as np
from jax.experimental import pallas as pl
from jax.experimental.pallas import tpu as pltpu
from jax.sharding import Mesh, PartitionSpec as P


def _round_up(a: int, m: int) -> int:
    return ((a + m - 1) // m) * m


def _agg_kernel(xt_ref, wst_ref, wnt_ref, bt_ref, src_ref, dst_ref,
                st_ref, z_ref, yt_ref):
    n_pad = z_ref.shape[1]
    h = st_ref.shape[0]
    te = src_ref.shape[1]

    @pl.when(pl.program_id(0) == 0)
    def _node_mm():
        tile_n = min(512, n_pad)
        for b in range(n_pad // tile_n):
            sl = pl.ds(b * tile_n, tile_n)
            xt = xt_ref[:, sl]
            st_ref[:, sl] = (
                jnp.dot(wst_ref[...], xt, preferred_element_type=jnp.float32)
                + bt_ref[...])
            y = jnp.dot(wnt_ref[...], xt, preferred_element_type=jnp.float32,
                        precision=jax.lax.Precision.HIGHEST)
            y_hi = y.astype(jnp.bfloat16).astype(jnp.float32)
            yt_ref[:h, sl] = y_hi
            yt_ref[h:, sl] = y - y_hi
        z_ref[...] = jnp.zeros_like(z_ref)

    node_iota = jax.lax.broadcasted_iota(jnp.int32, (n_pad, te), 0)
    oh_src = (node_iota == src_ref[...]).astype(jnp.float32)
    oh_dst = (node_iota == dst_ref[...]).astype(jnp.float32)
    p = jnp.dot(yt_ref[...], oh_src, preferred_element_type=jnp.float32)
    z_ref[...] += jax.lax.dot_general(
        p, oh_dst, (((1,), (1,)), ((), ())),
        preferred_element_type=jnp.float32)


def _run_agg(xt, wst, wnt, bt, src2d, dst2d, *, tile_e):
    d_in, n_pad = xt.shape
    h = wst.shape[0]
    e_pad = src2d.shape[1]
    return pl.pallas_call(
        _agg_kernel,
        out_shape=(jax.ShapeDtypeStruct((h, n_pad), jnp.float32),
                   jax.ShapeDtypeStruct((2 * h, n_pad), jnp.float32)),
        grid_spec=pltpu.PrefetchScalarGridSpec(
            num_scalar_prefetch=0,
            grid=(e_pad // tile_e,),
            in_specs=[
                pl.BlockSpec((d_in, n_pad), lambda j: (0, 0)),
                pl.BlockSpec((h, d_in), lambda j: (0, 0)),
                pl.BlockSpec((h, d_in), lambda j: (0, 0)),
                pl.BlockSpec((h, 1), lambda j: (0, 0)),
                pl.BlockSpec((1, tile_e), lambda j: (0, j)),
                pl.BlockSpec((1, tile_e), lambda j: (0, j)),
            ],
            out_specs=(pl.BlockSpec((h, n_pad), lambda j: (0, 0)),
                       pl.BlockSpec((2 * h, n_pad), lambda j: (0, 0))),
            scratch_shapes=[pltpu.VMEM((2 * h, n_pad), jnp.float32)],
        ),
        compiler_params=pltpu.CompilerParams(
            dimension_semantics=("arbitrary",)),
    )(xt, wst, wnt, bt, src2d, dst2d)


_LO = 256


def _edge_kernel(st_ref, z_ref, w1t_ref, b1t_ref, w2_ref, b2_ref,
                 src_ref, dst_ref, wep_ref, we2_ref, be2_ref,
                 out_ref, f3_ref):
    n_pad = st_ref.shape[1]
    n_hi = n_pad // _LO
    te = src_ref.shape[1]

    h = st_ref.shape[0]

    @pl.when(pl.program_id(0) == 0)
    def _node_post():
        for b in range(n_pad // 128):
            sl = pl.ds(b * 128, 128)
            emb = jnp.tanh(st_ref[:, sl] + z_ref[:h, sl] + z_ref[h:, sl])
            hphi = jnp.maximum(
                jnp.dot(w1t_ref[...], emb, preferred_element_type=jnp.float32)
                + b1t_ref[...], 0.0)
            phin = (jnp.sum(w2_ref[...] * hphi, axis=0, keepdims=True)
                    + b2_ref[...])
            phin_hi = phin.astype(jnp.bfloat16).astype(jnp.float32)
            hi, lo = divmod(b * 128, _LO)
            lsl = pl.ds(lo, 128)
            f3_ref[8 * hi + 0:8 * hi + 3, lsl] = emb[0:3, :]
            f3_ref[8 * hi + 3:8 * hi + 4, lsl] = phin_hi
            f3_ref[8 * hi + 4:8 * hi + 5, lsl] = phin - phin_hi
            f3_ref[8 * hi + 5:8 * hi + 8, lsl] = jnp.zeros((3, 128),
                                                           jnp.float32)

    lane_iota = jax.lax.broadcasted_iota(jnp.int32, (_LO, te), 0)
    f3 = f3_ref[...]

    def gather(idx):
        lo = jnp.bitwise_and(idx, _LO - 1)
        hi = jnp.right_shift(idx, 8)
        ohlo = (lane_iota == lo).astype(jnp.float32)
        g = jnp.dot(f3, ohlo, preferred_element_type=jnp.float32)
        acc = g[0:8, :] * (hi == 0).astype(jnp.float32)
        for k in range(1, n_hi):
            acc = acc + g[8 * k:8 * (k + 1), :] * (hi == k).astype(jnp.float32)
        return acc

    fi = gather(src_ref[...])
    fj = gather(dst_ref[...])
    dt = fj[0:1, :] - fi[0:1, :]
    dx = fj[1:2, :] - fi[1:2, :]
    dy = fj[2:3, :] - fi[2:3, :]
    phi = fi[3:4, :] + fi[4:5, :]
    dx2 = dx * dx + dy * dy
    dt2 = dt * dt
    ds2 = -phi * dt2 + dx2
    spatial_dist = jnp.sqrt(dx2)
    is_timelike = (ds2 < 0.0).astype(jnp.float32)
    ones = jnp.ones((1, te), jnp.float32)
    zeros = jnp.zeros((1, te), jnp.float32)
    feats = jnp.concatenate(
        [dx, dy, dt, ds2, spatial_dist, is_timelike, ones, zeros], axis=0)
    he = jnp.maximum(
        jnp.dot(wep_ref[...], feats, preferred_element_type=jnp.float32), 0.0)
    logit = (jnp.sum(we2_ref[...] * he, axis=0, keepdims=True) + be2_ref[...])
    out_ref[...] = jax.nn.sigmoid(logit)


def _run_edge(st, z, w1t, b1t, w2, b2, src2d, dst2d, wep, we2, be2, *, tile_e):
    h, n_pad = st.shape
    d_phi = w1t.shape[0]
    d_ep = wep.shape[0]
    e_pad = src2d.shape[1]
    n_hi = n_pad // _LO

    def full(shape):
        return pl.BlockSpec(shape, lambda i: tuple(0 for _ in shape))

    return pl.pallas_call(
        _edge_kernel,
        out_shape=jax.ShapeDtypeStruct((1, e_pad), jnp.float32),
        grid_spec=pltpu.PrefetchScalarGridSpec(
            num_scalar_prefetch=0,
            grid=(e_pad // tile_e,),
            in_specs=[
                full((h, n_pad)), full((2 * h, n_pad)),
                full((d_phi, h)), full((d_phi, 1)),
                full((d_phi, 1)), full((1, 1)),
                pl.BlockSpec((1, tile_e), lambda i: (0, i)),
                pl.BlockSpec((1, tile_e), lambda i: (0, i)),
                full((d_ep, 8)), full((d_ep, 1)), full((1, 1)),
            ],
            out_specs=pl.BlockSpec((1, tile_e), lambda i: (0, i)),
            scratch_shapes=[pltpu.VMEM((n_hi * 8, _LO), jnp.float32)],
        ),
        compiler_params=pltpu.CompilerParams(
            dimension_semantics=("arbitrary",)),
    )(st, z, w1t, b1t, w2, b2, src2d, dst2d, wep, we2, be2)


_TE_AGG = 1024
_TE_EDGE = 4096


def _forward(x, edge_index, gnn_w_self, gnn_w_nbr, gnn_b,
             phi_w1, phi_b1, phi_w2, phi_b2,
             ep_w1, ep_b1, ep_w2, ep_b2, axis_name=None):
    n, d_in = x.shape
    src = edge_index[0].astype(jnp.int32)
    dst = edge_index[1].astype(jnp.int32)
    e = src.shape[0]

    n_pad = _round_up(max(n, _LO), _LO)
    xt = x.T
    if n_pad != n:
        xt = jnp.zeros((d_in, n_pad), jnp.float32).at[:, :n].set(xt)

    e_pad = _round_up(max(e, _TE_EDGE), _TE_EDGE)
    src_p = jnp.full((1, e_pad), n_pad, jnp.int32).at[0, :e].set(src)
    dst_p = jnp.full((1, e_pad), n_pad, jnp.int32).at[0, :e].set(dst)

    st, z = _run_agg(xt, gnn_w_self.T, gnn_w_nbr.T, gnn_b.T, src_p, dst_p,
                     tile_e=_TE_AGG)
    if axis_name is not None:
        z = jax.lax.psum(z, axis_name)

    wep = jnp.concatenate(
        [ep_w1.T, ep_b1.T, jnp.zeros((ep_w1.shape[1], 1), jnp.float32)], axis=1)
    probs = _run_edge(st, z, phi_w1.T, phi_b1.T, phi_w2, phi_b2,
                      src_p, dst_p, wep, ep_w2, ep_b2, tile_e=_TE_EDGE)
    return probs[:, :e]


def kernel(x, edge_index, gnn_w_self, gnn_w_nbr, gnn_b,
           phi_w1, phi_b1, phi_w2, phi_b2,
           ep_w1, ep_b1, ep_w2, ep_b2):
    e = edge_index.shape[1]
    args = (x, edge_index, gnn_w_self, gnn_w_nbr, gnn_b,
            phi_w1, phi_b1, phi_w2, phi_b2, ep_w1, ep_b1, ep_w2, ep_b2)
    devs = jax.devices()
    if len(devs) >= 2 and e % 2 == 0:
        mesh = Mesh(np.asarray(devs[:2]), ("c",))
        in_specs = (P(), P(None, "c")) + (P(),) * 11
        fwd = jax.shard_map(
            functools.partial(_forward, axis_name="c"),
            mesh=mesh, in_specs=in_specs, out_specs=P(None, "c"),
            check_vma=False)
        probs = fwd(*args)
    else:
        probs = _forward(*args)
    return probs[0, :e]
